# baseline (speedup 1.0000x reference)
import functools

import jax
import jax.numpy as jnp
import numpy as np
from jax import lax
from jax.experimental import pallas as pl
from jax.experimental.pallas import tpu as pltpu
from jax.experimental.pallas import tpu_sc as plsc

VOCAB_DIM = 1000000
DIM = 64
L_SEQ = 200
L_PAD = 256
CHUNKS = ((0, 128), (128, 72))
EPS = 1e-5
CONV_BLK = 2048

_PERM = np.concatenate([
    np.arange(0, 32, 2), np.arange(1, 32, 2),
    np.arange(32, 64, 2), np.arange(33, 64, 2),
])


def _convert_kernel(tT_ref, out_ref):
    a = tT_ref[...]
    at = jnp.transpose(a, (1, 0))
    bf = at.astype(jnp.bfloat16)
    out_ref[...] = jnp.concatenate(
        [bf, jnp.zeros((CONV_BLK, DIM), jnp.bfloat16)], axis=-1)


def _convert_table(tT):
    grid = (VOCAB_DIM + CONV_BLK - 1) // CONV_BLK
    return pl.pallas_call(
        _convert_kernel,
        grid=(grid,),
        in_specs=[pl.BlockSpec((DIM, CONV_BLK), lambda i: (0, i))],
        out_specs=pl.BlockSpec((CONV_BLK, 2 * DIM), lambda i: (i, 0)),
        out_shape=jax.ShapeDtypeStruct((VOCAB_DIM, 2 * DIM), jnp.bfloat16),
    )(tT)


def _make_pooling(B):
    info = plsc.get_sparse_core_info()
    NC, NS = info.num_cores, info.num_subcores
    NW = NC * NS
    assert B % NW == 0
    b_per_w = B // NW
    assert b_per_w % 2 == 0
    mesh = plsc.VectorSubcoreMesh(core_axis_name="c", subcore_axis_name="s")

    @functools.partial(
        pl.kernel,
        mesh=mesh,
        compiler_params=pltpu.CompilerParams(
            use_tc_tiling_on_sc=False, needs_layout_passes=False),
        out_type=jax.ShapeDtypeStruct((B, DIM), jnp.float32),
        scratch_types=[
            pltpu.VMEM((b_per_w, L_PAD), jnp.int32),
            pltpu.VMEM((2, L_SEQ, 2 * DIM), jnp.bfloat16),
            pltpu.VMEM((b_per_w, DIM), jnp.float32),
            pltpu.SemaphoreType.DMA,
            pltpu.SemaphoreType.DMA,
        ],
    )
    def pool_kernel(x_hbm, table_hbm, out_hbm, idx_v, rows_v, pooled_v,
                    sem0, sem1):
        wid = lax.axis_index("s") * NC + lax.axis_index("c")
        pltpu.sync_copy(x_hbm.at[wid], idx_v)

        sems = (sem0, sem1)

        def issue(r, b):
            for off, n in CHUNKS:
                pltpu.make_async_copy(
                    table_hbm.at[idx_v.at[r, pl.ds(off, n)]],
                    rows_v.at[b, pl.ds(off, n)],
                    sems[b],
                ).start()

        def wait(r, b):
            for off, n in CHUNKS:
                pltpu.make_async_copy(
                    table_hbm.at[idx_v.at[r, pl.ds(off, n)]],
                    rows_v.at[b, pl.ds(off, n)],
                    sems[b],
                ).wait()

        def reduce_row(r, b):
            rows = rows_v.at[b]

            def body(j, accs):
                w0 = rows[j, pl.ds(0, 32)]
                w1 = rows[j, pl.ds(32, 32)]
                a0, b0 = plsc.unpack(
                    w0, format=plsc.PackFormat.INTERLEAVED,
                    preferred_element_type=jnp.float32)
                a1, b1 = plsc.unpack(
                    w1, format=plsc.PackFormat.INTERLEAVED,
                    preferred_element_type=jnp.float32)
                return (accs[0] + a0, accs[1] + b0, accs[2] + a1,
                        accs[3] + b1)

            init = tuple(jnp.zeros((16,), jnp.float32) for _ in range(4))
            accs = lax.fori_loop(0, L_SEQ, body, init, unroll=8)
            scale = jnp.float32(1.0 / L_SEQ)
            for k in range(4):
                pooled_v[r, pl.ds(k * 16, 16)] = accs[k] * scale

        issue(0, 0)

        def outer(g, carry):
            for b in range(2):
                r = g * 2 + b
                nxt = r + 1

                @pl.when(nxt < b_per_w)
                def _():
                    issue(nxt, 1 - b)

                wait(r, b)
                reduce_row(r, b)
            return carry

        lax.fori_loop(0, b_per_w // 2, outer, 0)
        pltpu.sync_copy(pooled_v, out_hbm.at[pl.ds(wid * b_per_w, b_per_w)])

    return pool_kernel


def _tail_kernel(p_ref, wt_ref, b_ref, g_ref, be_ref, o_ref):
    p = p_ref[...]
    h = jnp.dot(p, wt_ref[...], preferred_element_type=jnp.float32)
    h = h + b_ref[...]
    n = jnp.float32(1.0 / p.shape[0])
    mu = jnp.sum(h, axis=0, keepdims=True) * n
    d = h - mu
    var = jnp.sum(d * d, axis=0, keepdims=True) * n
    hn = d * lax.rsqrt(var + EPS) * g_ref[...] + be_ref[...]
    m = jnp.float32(1.0 / p.shape[1])
    mu2 = jnp.sum(hn, axis=1, keepdims=True) * m
    d2 = hn - mu2
    var2 = jnp.sum(d2 * d2, axis=1, keepdims=True) * m
    o_ref[...] = d2 * lax.rsqrt(var2 + EPS)


@jax.jit
def kernel(x, table, W, b, gamma, beta):
    B = x.shape[0]
    info = plsc.get_sparse_core_info()
    NW = info.num_cores * info.num_subcores
    x_pad = jnp.pad(x, ((0, 0), (0, L_PAD - L_SEQ)))
    x_blocks = x_pad.reshape(NW, B // NW, L_PAD)
    table_bf = _convert_table(table.T)
    pooled = _make_pooling(B)(x_blocks, table_bf)
    wt_perm = W.T[_PERM, :]
    return pl.pallas_call(
        _tail_kernel,
        out_shape=jax.ShapeDtypeStruct((B, DIM), jnp.float32),
    )(pooled, wt_perm, b.reshape(1, DIM), gamma.reshape(1, DIM),
      beta.reshape(1, DIM))


# speedup vs baseline: 1.9753x; 1.9753x over previous
import functools

import jax
import jax.numpy as jnp
from jax import lax
from jax.experimental import pallas as pl
from jax.experimental.pallas import tpu as pltpu
from jax.experimental.pallas import tpu_sc as plsc

VOCAB_DIM = 1000000
DIM = 64
L_SEQ = 200
L_PAD = 256
CHUNKS = ((0, 128), (128, 72))
EPS = 1e-5
CONV_BLK = 2048


def _convert_kernel(tT_ref, out_ref):
    a = tT_ref[...].astype(jnp.bfloat16)
    row = lax.broadcasted_iota(jnp.int32, (DIM, DIM), 0)
    col = lax.broadcasted_iota(jnp.int32, (DIM, DIM), 1)
    ident = (row == col).astype(jnp.bfloat16)
    at = lax.dot_general(a, ident, (((0,), (0,)), ((), ())),
                         preferred_element_type=jnp.float32)
    out_ref[:, :DIM] = at


def _convert_table(tT):
    grid = (VOCAB_DIM + CONV_BLK - 1) // CONV_BLK
    return pl.pallas_call(
        _convert_kernel,
        grid=(grid,),
        in_specs=[pl.BlockSpec((DIM, CONV_BLK), lambda i: (0, i))],
        out_specs=pl.BlockSpec((CONV_BLK, 2 * DIM), lambda i: (i, 0)),
        out_shape=jax.ShapeDtypeStruct((VOCAB_DIM, 2 * DIM), jnp.float32),
    )(tT)


def _make_pooling(B):
    info = plsc.get_sparse_core_info()
    NC, NS = info.num_cores, info.num_subcores
    NW = NC * NS
    assert B % NW == 0
    b_per_w = B // NW
    assert b_per_w % 2 == 0
    mesh = plsc.VectorSubcoreMesh(core_axis_name="c", subcore_axis_name="s")

    @functools.partial(
        pl.kernel,
        mesh=mesh,
        compiler_params=pltpu.CompilerParams(use_tc_tiling_on_sc=True),
        out_type=jax.ShapeDtypeStruct((B, DIM), jnp.float32),
        scratch_types=[
            pltpu.VMEM((b_per_w, L_PAD), jnp.int32),
            pltpu.VMEM((2, L_SEQ, 2 * DIM), jnp.float32),
            pltpu.VMEM((b_per_w, DIM), jnp.float32),
            pltpu.SemaphoreType.DMA,
            pltpu.SemaphoreType.DMA,
        ],
    )
    def pool_kernel(x_hbm, table_hbm, out_hbm, idx_v, rows_v, pooled_v,
                    sem0, sem1):
        wid = lax.axis_index("s") * NC + lax.axis_index("c")
        pltpu.sync_copy(x_hbm.at[wid], idx_v)

        sems = (sem0, sem1)

        def issue(r, b):
            for off, n in CHUNKS:
                pltpu.make_async_copy(
                    table_hbm.at[idx_v.at[r, pl.ds(off, n)]],
                    rows_v.at[b, pl.ds(off, n)],
                    sems[b],
                ).start()

        def wait(r, b):
            for off, n in CHUNKS:
                pltpu.make_async_copy(
                    table_hbm.at[idx_v.at[r, pl.ds(off, n)]],
                    rows_v.at[b, pl.ds(off, n)],
                    sems[b],
                ).wait()

        def reduce_row(r, b):
            rows = rows_v.at[b]

            def body(j, accs):
                return tuple(
                    accs[k] + rows[j, pl.ds(k * 16, 16)] for k in range(4)
                )

            init = tuple(jnp.zeros((16,), jnp.float32) for _ in range(4))
            accs = lax.fori_loop(0, L_SEQ, body, init, unroll=8)
            scale = jnp.float32(1.0 / L_SEQ)
            for k in range(4):
                pooled_v[r, pl.ds(k * 16, 16)] = accs[k] * scale

        issue(0, 0)

        def outer(g, carry):
            for b in range(2):
                r = g * 2 + b
                nxt = r + 1

                @pl.when(nxt < b_per_w)
                def _():
                    issue(nxt, 1 - b)

                wait(r, b)
                reduce_row(r, b)
            return carry

        lax.fori_loop(0, b_per_w // 2, outer, 0)
        pltpu.sync_copy(pooled_v, out_hbm.at[pl.ds(wid * b_per_w, b_per_w)])

    return pool_kernel


def _tail_kernel(p_ref, wt_ref, b_ref, g_ref, be_ref, o_ref):
    p = p_ref[...]
    h = jnp.dot(p, wt_ref[...], preferred_element_type=jnp.float32)
    h = h + b_ref[...]
    n = jnp.float32(1.0 / p.shape[0])
    mu = jnp.sum(h, axis=0, keepdims=True) * n
    d = h - mu
    var = jnp.sum(d * d, axis=0, keepdims=True) * n
    hn = d * lax.rsqrt(var + EPS) * g_ref[...] + be_ref[...]
    m = jnp.float32(1.0 / p.shape[1])
    mu2 = jnp.sum(hn, axis=1, keepdims=True) * m
    d2 = hn - mu2
    var2 = jnp.sum(d2 * d2, axis=1, keepdims=True) * m
    o_ref[...] = d2 * lax.rsqrt(var2 + EPS)


@jax.jit
def kernel(x, table, W, b, gamma, beta):
    B = x.shape[0]
    info = plsc.get_sparse_core_info()
    NW = info.num_cores * info.num_subcores
    x_pad = jnp.pad(x, ((0, 0), (0, L_PAD - L_SEQ)))
    x_blocks = x_pad.reshape(NW, B // NW, L_PAD)
    table_wide = _convert_table(table.T)
    pooled = _make_pooling(B)(x_blocks, table_wide)
    return pl.pallas_call(
        _tail_kernel,
        out_shape=jax.ShapeDtypeStruct((B, DIM), jnp.float32),
    )(pooled, W.T, b.reshape(1, DIM), gamma.reshape(1, DIM),
      beta.reshape(1, DIM))


# speedup vs baseline: 2.8472x; 1.4414x over previous
import functools

import jax
import jax.numpy as jnp
from jax import lax
from jax.experimental import pallas as pl
from jax.experimental.pallas import tpu as pltpu
from jax.experimental.pallas import tpu_sc as plsc

VOCAB_DIM = 1000000
DIM = 64
L_SEQ = 200
L_PAD = 256
CHUNKS = ((0, 128), (128, 72))
EPS = 1e-5
CONV_BLK = 8192


def _convert_kernel(tT_ref, out_ref):
    a = tT_ref[...].astype(jnp.bfloat16)
    row = lax.broadcasted_iota(jnp.int32, (DIM, DIM), 0)
    col = lax.broadcasted_iota(jnp.int32, (DIM, DIM), 1)
    ident = (row == col).astype(jnp.bfloat16)
    at = lax.dot_general(a, ident, (((0,), (0,)), ((), ())),
                         preferred_element_type=jnp.float32)
    out_ref[:, :DIM] = at


def _convert_table(tT):
    grid = (VOCAB_DIM + CONV_BLK - 1) // CONV_BLK
    return pl.pallas_call(
        _convert_kernel,
        grid=(grid,),
        in_specs=[pl.BlockSpec((DIM, CONV_BLK), lambda i: (0, i))],
        out_specs=pl.BlockSpec((CONV_BLK, 2 * DIM), lambda i: (i, 0)),
        out_shape=jax.ShapeDtypeStruct((VOCAB_DIM, 2 * DIM), jnp.float32),
        compiler_params=pltpu.CompilerParams(
            dimension_semantics=("arbitrary",)),
    )(tT)


def _make_pooling(B):
    info = plsc.get_sparse_core_info()
    NC, NS = info.num_cores, info.num_subcores
    NW = NC * NS
    assert B % NW == 0
    b_per_w = B // NW
    assert b_per_w % 2 == 0
    mesh = plsc.VectorSubcoreMesh(core_axis_name="c", subcore_axis_name="s")

    @functools.partial(
        pl.kernel,
        mesh=mesh,
        compiler_params=pltpu.CompilerParams(use_tc_tiling_on_sc=True),
        out_type=jax.ShapeDtypeStruct((B, DIM), jnp.float32),
        scratch_types=[
            pltpu.VMEM((b_per_w, L_PAD), jnp.int32),
            pltpu.VMEM((2, L_SEQ, 2 * DIM), jnp.float32),
            pltpu.VMEM((b_per_w, DIM), jnp.float32),
            pltpu.SemaphoreType.DMA,
            pltpu.SemaphoreType.DMA,
        ],
    )
    def pool_kernel(x_hbm, table_hbm, out_hbm, idx_v, rows_v, pooled_v,
                    sem0, sem1):
        wid = lax.axis_index("s") * NC + lax.axis_index("c")
        pltpu.sync_copy(x_hbm.at[wid], idx_v)

        sems = (sem0, sem1)

        def issue(r, b):
            for off, n in CHUNKS:
                pltpu.make_async_copy(
                    table_hbm.at[idx_v.at[r, pl.ds(off, n)]],
                    rows_v.at[b, pl.ds(off, n)],
                    sems[b],
                ).start()

        def wait(r, b):
            for off, n in CHUNKS:
                pltpu.make_async_copy(
                    table_hbm.at[idx_v.at[r, pl.ds(off, n)]],
                    rows_v.at[b, pl.ds(off, n)],
                    sems[b],
                ).wait()

        def reduce_row(r, b):
            rows = rows_v.at[b]

            def body(j, accs):
                return tuple(
                    accs[k] + rows[j, pl.ds(k * 16, 16)] for k in range(4)
                )

            init = tuple(jnp.zeros((16,), jnp.float32) for _ in range(4))
            accs = lax.fori_loop(0, L_SEQ, body, init, unroll=8)
            scale = jnp.float32(1.0 / L_SEQ)
            for k in range(4):
                pooled_v[r, pl.ds(k * 16, 16)] = accs[k] * scale

        issue(0, 0)

        def outer(g, carry):
            for b in range(2):
                r = g * 2 + b
                nxt = r + 1

                @pl.when(nxt < b_per_w)
                def _():
                    issue(nxt, 1 - b)

                wait(r, b)
                reduce_row(r, b)
            return carry

        lax.fori_loop(0, b_per_w // 2, outer, 0)
        pltpu.sync_copy(pooled_v, out_hbm.at[pl.ds(wid * b_per_w, b_per_w)])

    return pool_kernel


def _tail_kernel(p_ref, wt_ref, b_ref, g_ref, be_ref, o_ref):
    p = p_ref[...]
    h = jnp.dot(p, wt_ref[...], preferred_element_type=jnp.float32)
    h = h + b_ref[...]
    n = jnp.float32(1.0 / p.shape[0])
    mu = jnp.sum(h, axis=0, keepdims=True) * n
    d = h - mu
    var = jnp.sum(d * d, axis=0, keepdims=True) * n
    hn = d * lax.rsqrt(var + EPS) * g_ref[...] + be_ref[...]
    m = jnp.float32(1.0 / p.shape[1])
    mu2 = jnp.sum(hn, axis=1, keepdims=True) * m
    d2 = hn - mu2
    var2 = jnp.sum(d2 * d2, axis=1, keepdims=True) * m
    o_ref[...] = d2 * lax.rsqrt(var2 + EPS)


@jax.jit
def kernel(x, table, W, b, gamma, beta):
    B = x.shape[0]
    info = plsc.get_sparse_core_info()
    NW = info.num_cores * info.num_subcores
    x_pad = jnp.pad(x, ((0, 0), (0, L_PAD - L_SEQ)))
    x_blocks = x_pad.reshape(NW, B // NW, L_PAD)
    table_wide = _convert_table(table.T)
    pooled = _make_pooling(B)(x_blocks, table_wide)
    return pl.pallas_call(
        _tail_kernel,
        out_shape=jax.ShapeDtypeStruct((B, DIM), jnp.float32),
    )(pooled, W.T, b.reshape(1, DIM), gamma.reshape(1, DIM),
      beta.reshape(1, DIM))


# speedup vs baseline: 2.9229x; 1.0266x over previous
import functools

import jax
import jax.numpy as jnp
from jax import lax
from jax.experimental import pallas as pl
from jax.experimental.pallas import tpu as pltpu
from jax.experimental.pallas import tpu_sc as plsc

VOCAB_DIM = 1000000
DIM = 64
L_SEQ = 200
L_PAD = 256
CHUNKS = ((0, 128), (128, 72))
EPS = 1e-5
CONV_BLK = 16384


def _convert_kernel(tT_ref, out_ref):
    a = tT_ref[...].astype(jnp.bfloat16)
    row = lax.broadcasted_iota(jnp.int32, (DIM, DIM), 0)
    col = lax.broadcasted_iota(jnp.int32, (DIM, DIM), 1)
    ident = (row == col).astype(jnp.bfloat16)
    at = lax.dot_general(a, ident, (((0,), (0,)), ((), ())),
                         preferred_element_type=jnp.float32)
    out_ref[:, :DIM] = at


def _convert_table(tT):
    grid = (VOCAB_DIM + CONV_BLK - 1) // CONV_BLK
    return pl.pallas_call(
        _convert_kernel,
        grid=(grid,),
        in_specs=[pl.BlockSpec((DIM, CONV_BLK), lambda i: (0, i))],
        out_specs=pl.BlockSpec((CONV_BLK, 2 * DIM), lambda i: (i, 0)),
        out_shape=jax.ShapeDtypeStruct((VOCAB_DIM, 2 * DIM), jnp.float32),
        compiler_params=pltpu.CompilerParams(
            dimension_semantics=("arbitrary",)),
    )(tT)


def _make_pooling(B):
    info = plsc.get_sparse_core_info()
    NC, NS = info.num_cores, info.num_subcores
    NW = NC * NS
    assert B % NW == 0
    b_per_w = B // NW
    assert b_per_w % 2 == 0
    mesh = plsc.VectorSubcoreMesh(core_axis_name="c", subcore_axis_name="s")

    @functools.partial(
        pl.kernel,
        mesh=mesh,
        compiler_params=pltpu.CompilerParams(use_tc_tiling_on_sc=True),
        out_type=jax.ShapeDtypeStruct((B, DIM), jnp.float32),
        scratch_types=[
            pltpu.VMEM((b_per_w, L_PAD), jnp.int32),
            pltpu.VMEM((2, L_SEQ, 2 * DIM), jnp.float32),
            pltpu.VMEM((b_per_w, DIM), jnp.float32),
            pltpu.SemaphoreType.DMA,
            pltpu.SemaphoreType.DMA,
        ],
    )
    def pool_kernel(x_hbm, table_hbm, out_hbm, idx_v, rows_v, pooled_v,
                    sem0, sem1):
        wid = lax.axis_index("s") * NC + lax.axis_index("c")
        pltpu.sync_copy(x_hbm.at[wid], idx_v)

        sems = (sem0, sem1)

        def issue(r, b):
            for off, n in CHUNKS:
                pltpu.make_async_copy(
                    table_hbm.at[idx_v.at[r, pl.ds(off, n)]],
                    rows_v.at[b, pl.ds(off, n)],
                    sems[b],
                ).start()

        def wait(r, b):
            for off, n in CHUNKS:
                pltpu.make_async_copy(
                    table_hbm.at[idx_v.at[r, pl.ds(off, n)]],
                    rows_v.at[b, pl.ds(off, n)],
                    sems[b],
                ).wait()

        def reduce_row(r, b):
            rows = rows_v.at[b]

            def body(j, accs):
                return tuple(
                    accs[k] + rows[j, pl.ds(k * 16, 16)] for k in range(4)
                )

            init = tuple(jnp.zeros((16,), jnp.float32) for _ in range(4))
            accs = lax.fori_loop(0, L_SEQ, body, init, unroll=8)
            scale = jnp.float32(1.0 / L_SEQ)
            for k in range(4):
                pooled_v[r, pl.ds(k * 16, 16)] = accs[k] * scale

        issue(0, 0)

        def outer(g, carry):
            for b in range(2):
                r = g * 2 + b
                nxt = r + 1

                @pl.when(nxt < b_per_w)
                def _():
                    issue(nxt, 1 - b)

                wait(r, b)
                reduce_row(r, b)
            return carry

        lax.fori_loop(0, b_per_w // 2, outer, 0)
        pltpu.sync_copy(pooled_v, out_hbm.at[pl.ds(wid * b_per_w, b_per_w)])

    return pool_kernel


def _tail_kernel(p_ref, wt_ref, b_ref, g_ref, be_ref, o_ref):
    p = p_ref[...]
    h = jnp.dot(p, wt_ref[...], preferred_element_type=jnp.float32)
    h = h + b_ref[...]
    n = jnp.float32(1.0 / p.shape[0])
    mu = jnp.sum(h, axis=0, keepdims=True) * n
    d = h - mu
    var = jnp.sum(d * d, axis=0, keepdims=True) * n
    hn = d * lax.rsqrt(var + EPS) * g_ref[...] + be_ref[...]
    m = jnp.float32(1.0 / p.shape[1])
    mu2 = jnp.sum(hn, axis=1, keepdims=True) * m
    d2 = hn - mu2
    var2 = jnp.sum(d2 * d2, axis=1, keepdims=True) * m
    o_ref[...] = d2 * lax.rsqrt(var2 + EPS)


@jax.jit
def kernel(x, table, W, b, gamma, beta):
    B = x.shape[0]
    info = plsc.get_sparse_core_info()
    NW = info.num_cores * info.num_subcores
    x_pad = jnp.pad(x, ((0, 0), (0, L_PAD - L_SEQ)))
    x_blocks = x_pad.reshape(NW, B // NW, L_PAD)
    table_wide = _convert_table(table.T)
    pooled = _make_pooling(B)(x_blocks, table_wide)
    return pl.pallas_call(
        _tail_kernel,
        out_shape=jax.ShapeDtypeStruct((B, DIM), jnp.float32),
    )(pooled, W.T, b.reshape(1, DIM), gamma.reshape(1, DIM),
      beta.reshape(1, DIM))


# speedup vs baseline: 3.0591x; 1.0466x over previous
import functools

import jax
import jax.numpy as jnp
from jax import lax
from jax.experimental import pallas as pl
from jax.experimental.pallas import tpu as pltpu
from jax.experimental.pallas import tpu_sc as plsc

VOCAB_DIM = 1000000
DIM = 64
L_SEQ = 200
L_PAD = 256
CHUNKS = ((0, 128), (128, 72))
EPS = 1e-5
CONV_BLK = 16384


def _convert_kernel(tT_ref, out_ref):
    a = tT_ref[...].astype(jnp.bfloat16)
    row = lax.broadcasted_iota(jnp.int32, (DIM, DIM), 0)
    col = lax.broadcasted_iota(jnp.int32, (DIM, DIM), 1)
    ident = (row == col).astype(jnp.bfloat16)
    at = lax.dot_general(a, ident, (((0,), (0,)), ((), ())),
                         preferred_element_type=jnp.float32)
    out_ref[:, :DIM] = at


def _convert_table(tT):
    grid = (VOCAB_DIM + CONV_BLK - 1) // CONV_BLK
    return pl.pallas_call(
        _convert_kernel,
        grid=(grid,),
        in_specs=[pl.BlockSpec((DIM, CONV_BLK), lambda i: (0, i))],
        out_specs=pl.BlockSpec((CONV_BLK, 2 * DIM), lambda i: (i, 0)),
        out_shape=jax.ShapeDtypeStruct((VOCAB_DIM, 2 * DIM), jnp.float32),
        compiler_params=pltpu.CompilerParams(
            dimension_semantics=("arbitrary",)),
    )(tT)


def _make_pooling(B):
    info = plsc.get_sparse_core_info()
    NC, NS = info.num_cores, info.num_subcores
    NW = NC * NS
    assert B % NW == 0
    b_per_w = B // NW
    assert b_per_w % 2 == 0
    mesh = plsc.VectorSubcoreMesh(core_axis_name="c", subcore_axis_name="s")

    @functools.partial(
        pl.kernel,
        mesh=mesh,
        compiler_params=pltpu.CompilerParams(use_tc_tiling_on_sc=True),
        out_type=jax.ShapeDtypeStruct((B, DIM), jnp.float32),
        scratch_types=[
            pltpu.VMEM((b_per_w, L_PAD), jnp.int32),
            pltpu.VMEM((3, L_SEQ, 2 * DIM), jnp.float32),
            pltpu.VMEM((b_per_w, DIM), jnp.float32),
            pltpu.SemaphoreType.DMA,
            pltpu.SemaphoreType.DMA,
            pltpu.SemaphoreType.DMA,
        ],
    )
    def pool_kernel(x_hbm, table_hbm, out_hbm, idx_v, rows_v, pooled_v,
                    sem0, sem1, sem2):
        wid = lax.axis_index("s") * NC + lax.axis_index("c")
        pltpu.sync_copy(x_hbm.at[wid], idx_v)

        sems = (sem0, sem1, sem2)

        def issue(r, b):
            for off, n in CHUNKS:
                pltpu.make_async_copy(
                    table_hbm.at[idx_v.at[r, pl.ds(off, n)]],
                    rows_v.at[b, pl.ds(off, n)],
                    sems[b],
                ).start()

        def wait(r, b):
            for off, n in CHUNKS:
                pltpu.make_async_copy(
                    table_hbm.at[idx_v.at[r, pl.ds(off, n)]],
                    rows_v.at[b, pl.ds(off, n)],
                    sems[b],
                ).wait()

        def reduce_row(r, b):
            rows = rows_v.at[b]

            def body(j, accs):
                return tuple(
                    accs[k] + rows[j, pl.ds(k * 16, 16)] for k in range(4)
                )

            init = tuple(jnp.zeros((16,), jnp.float32) for _ in range(4))
            accs = lax.fori_loop(0, L_SEQ, body, init, unroll=8)
            scale = jnp.float32(1.0 / L_SEQ)
            for k in range(4):
                pooled_v[r, pl.ds(k * 16, 16)] = accs[k] * scale

        issue(0, 0)
        issue(1, 1)
        n_main = (b_per_w - 2) // 3

        def outer(g, carry):
            for b in range(3):
                r = g * 3 + b
                issue(r + 2, (b + 2) % 3)
                wait(r, b)
                reduce_row(r, b)
            return carry

        lax.fori_loop(0, n_main, outer, 0)
        for r in range(3 * n_main, b_per_w):
            wait(r, r % 3)
            reduce_row(r, r % 3)
        pltpu.sync_copy(pooled_v, out_hbm.at[pl.ds(wid * b_per_w, b_per_w)])

    return pool_kernel


def _tail_kernel(p_ref, wt_ref, b_ref, g_ref, be_ref, o_ref):
    p = p_ref[...]
    h = jnp.dot(p, wt_ref[...], preferred_element_type=jnp.float32)
    h = h + b_ref[...]
    n = jnp.float32(1.0 / p.shape[0])
    mu = jnp.sum(h, axis=0, keepdims=True) * n
    d = h - mu
    var = jnp.sum(d * d, axis=0, keepdims=True) * n
    hn = d * lax.rsqrt(var + EPS) * g_ref[...] + be_ref[...]
    m = jnp.float32(1.0 / p.shape[1])
    mu2 = jnp.sum(hn, axis=1, keepdims=True) * m
    d2 = hn - mu2
    var2 = jnp.sum(d2 * d2, axis=1, keepdims=True) * m
    o_ref[...] = d2 * lax.rsqrt(var2 + EPS)


@jax.jit
def kernel(x, table, W, b, gamma, beta):
    B = x.shape[0]
    info = plsc.get_sparse_core_info()
    NW = info.num_cores * info.num_subcores
    x_pad = jnp.pad(x, ((0, 0), (0, L_PAD - L_SEQ)))
    x_blocks = x_pad.reshape(NW, B // NW, L_PAD)
    table_wide = _convert_table(table.T)
    pooled = _make_pooling(B)(x_blocks, table_wide)
    return pl.pallas_call(
        _tail_kernel,
        out_shape=jax.ShapeDtypeStruct((B, DIM), jnp.float32),
    )(pooled, W.T, b.reshape(1, DIM), gamma.reshape(1, DIM),
      beta.reshape(1, DIM))


# speedup vs baseline: 3.1041x; 1.0147x over previous
import functools

import jax
import jax.numpy as jnp
from jax import lax
from jax.experimental import pallas as pl
from jax.experimental.pallas import tpu as pltpu
from jax.experimental.pallas import tpu_sc as plsc

VOCAB_DIM = 1000000
DIM = 64
L_SEQ = 200
L_PAD = 256
CHUNKS = ((0, 128), (128, 72))
EPS = 1e-5
CONV_BLK = 32768


def _convert_kernel(tT_ref, out_ref):
    a = tT_ref[...].astype(jnp.bfloat16)
    row = lax.broadcasted_iota(jnp.int32, (DIM, DIM), 0)
    col = lax.broadcasted_iota(jnp.int32, (DIM, DIM), 1)
    ident = (row == col).astype(jnp.bfloat16)
    at = lax.dot_general(a, ident, (((0,), (0,)), ((), ())),
                         preferred_element_type=jnp.float32)
    out_ref[:, :DIM] = at


def _convert_table(tT):
    grid = (VOCAB_DIM + CONV_BLK - 1) // CONV_BLK
    return pl.pallas_call(
        _convert_kernel,
        grid=(grid,),
        in_specs=[pl.BlockSpec((DIM, CONV_BLK), lambda i: (0, i))],
        out_specs=pl.BlockSpec((CONV_BLK, 2 * DIM), lambda i: (i, 0)),
        out_shape=jax.ShapeDtypeStruct((VOCAB_DIM, 2 * DIM), jnp.float32),
        compiler_params=pltpu.CompilerParams(
            dimension_semantics=("arbitrary",)),
    )(tT)


def _make_pooling(B):
    info = plsc.get_sparse_core_info()
    NC, NS = info.num_cores, info.num_subcores
    NW = NC * NS
    assert B % NW == 0
    b_per_w = B // NW
    assert b_per_w % 2 == 0
    mesh = plsc.VectorSubcoreMesh(core_axis_name="c", subcore_axis_name="s")

    @functools.partial(
        pl.kernel,
        mesh=mesh,
        compiler_params=pltpu.CompilerParams(use_tc_tiling_on_sc=True),
        out_type=jax.ShapeDtypeStruct((B, DIM), jnp.float32),
        scratch_types=[
            pltpu.VMEM((b_per_w, L_PAD), jnp.int32),
            pltpu.VMEM((3, L_SEQ, 2 * DIM), jnp.float32),
            pltpu.VMEM((b_per_w, DIM), jnp.float32),
            pltpu.SemaphoreType.DMA,
            pltpu.SemaphoreType.DMA,
            pltpu.SemaphoreType.DMA,
        ],
    )
    def pool_kernel(x_hbm, table_hbm, out_hbm, idx_v, rows_v, pooled_v,
                    sem0, sem1, sem2):
        wid = lax.axis_index("s") * NC + lax.axis_index("c")
        pltpu.sync_copy(x_hbm.at[wid], idx_v)

        sems = (sem0, sem1, sem2)

        def issue(r, b):
            for off, n in CHUNKS:
                pltpu.make_async_copy(
                    table_hbm.at[idx_v.at[r, pl.ds(off, n)]],
                    rows_v.at[b, pl.ds(off, n)],
                    sems[b],
                ).start()

        def wait(r, b):
            for off, n in CHUNKS:
                pltpu.make_async_copy(
                    table_hbm.at[idx_v.at[r, pl.ds(off, n)]],
                    rows_v.at[b, pl.ds(off, n)],
                    sems[b],
                ).wait()

        def reduce_row(r, b):
            rows = rows_v.at[b]

            def body(j, accs):
                return tuple(
                    accs[k] + rows[j, pl.ds(k * 16, 16)] for k in range(4)
                )

            init = tuple(jnp.zeros((16,), jnp.float32) for _ in range(4))
            accs = lax.fori_loop(0, L_SEQ, body, init, unroll=8)
            scale = jnp.float32(1.0 / L_SEQ)
            for k in range(4):
                pooled_v[r, pl.ds(k * 16, 16)] = accs[k] * scale

        issue(0, 0)
        issue(1, 1)
        n_main = (b_per_w - 2) // 3

        def outer(g, carry):
            for b in range(3):
                r = g * 3 + b
                issue(r + 2, (b + 2) % 3)
                wait(r, b)
                reduce_row(r, b)
            return carry

        lax.fori_loop(0, n_main, outer, 0)
        for r in range(3 * n_main, b_per_w):
            wait(r, r % 3)
            reduce_row(r, r % 3)
        pltpu.sync_copy(pooled_v, out_hbm.at[pl.ds(wid * b_per_w, b_per_w)])

    return pool_kernel


def _tail_kernel(p_ref, wt_ref, b_ref, g_ref, be_ref, o_ref):
    p = p_ref[...]
    h = jnp.dot(p, wt_ref[...], preferred_element_type=jnp.float32)
    h = h + b_ref[...]
    n = jnp.float32(1.0 / p.shape[0])
    mu = jnp.sum(h, axis=0, keepdims=True) * n
    d = h - mu
    var = jnp.sum(d * d, axis=0, keepdims=True) * n
    hn = d * lax.rsqrt(var + EPS) * g_ref[...] + be_ref[...]
    m = jnp.float32(1.0 / p.shape[1])
    mu2 = jnp.sum(hn, axis=1, keepdims=True) * m
    d2 = hn - mu2
    var2 = jnp.sum(d2 * d2, axis=1, keepdims=True) * m
    o_ref[...] = d2 * lax.rsqrt(var2 + EPS)


@jax.jit
def kernel(x, table, W, b, gamma, beta):
    B = x.shape[0]
    info = plsc.get_sparse_core_info()
    NW = info.num_cores * info.num_subcores
    x_pad = jnp.pad(x, ((0, 0), (0, L_PAD - L_SEQ)))
    x_blocks = x_pad.reshape(NW, B // NW, L_PAD)
    table_wide = _convert_table(table.T)
    pooled = _make_pooling(B)(x_blocks, table_wide)
    return pl.pallas_call(
        _tail_kernel,
        out_shape=jax.ShapeDtypeStruct((B, DIM), jnp.float32),
    )(pooled, W.T, b.reshape(1, DIM), gamma.reshape(1, DIM),
      beta.reshape(1, DIM))


# speedup vs baseline: 3.1051x; 1.0003x over previous
import functools

import jax
import jax.numpy as jnp
from jax import lax
from jax.experimental import pallas as pl
from jax.experimental.pallas import tpu as pltpu
from jax.experimental.pallas import tpu_sc as plsc

VOCAB_DIM = 1000000
DIM = 64
L_SEQ = 200
L_PAD = 256
CHUNKS = ((0, 64), (64, 64), (128, 64), (192, 8))
EPS = 1e-5
CONV_BLK = 32768


def _convert_kernel(tT_ref, out_ref):
    a = tT_ref[...].astype(jnp.bfloat16)
    row = lax.broadcasted_iota(jnp.int32, (DIM, DIM), 0)
    col = lax.broadcasted_iota(jnp.int32, (DIM, DIM), 1)
    ident = (row == col).astype(jnp.bfloat16)
    at = lax.dot_general(a, ident, (((0,), (0,)), ((), ())),
                         preferred_element_type=jnp.float32)
    out_ref[:, :DIM] = at


def _convert_table(tT):
    grid = (VOCAB_DIM + CONV_BLK - 1) // CONV_BLK
    return pl.pallas_call(
        _convert_kernel,
        grid=(grid,),
        in_specs=[pl.BlockSpec((DIM, CONV_BLK), lambda i: (0, i))],
        out_specs=pl.BlockSpec((CONV_BLK, 2 * DIM), lambda i: (i, 0)),
        out_shape=jax.ShapeDtypeStruct((VOCAB_DIM, 2 * DIM), jnp.float32),
        compiler_params=pltpu.CompilerParams(
            dimension_semantics=("arbitrary",)),
    )(tT)


def _make_pooling(B):
    info = plsc.get_sparse_core_info()
    NC, NS = info.num_cores, info.num_subcores
    NW = NC * NS
    assert B % NW == 0
    b_per_w = B // NW
    assert b_per_w % 2 == 0
    mesh = plsc.VectorSubcoreMesh(core_axis_name="c", subcore_axis_name="s")

    @functools.partial(
        pl.kernel,
        mesh=mesh,
        compiler_params=pltpu.CompilerParams(use_tc_tiling_on_sc=True),
        out_type=jax.ShapeDtypeStruct((B, DIM), jnp.float32),
        scratch_types=[
            pltpu.VMEM((b_per_w, L_PAD), jnp.int32),
            pltpu.VMEM((3, L_SEQ, 2 * DIM), jnp.float32),
            pltpu.VMEM((b_per_w, DIM), jnp.float32),
            pltpu.SemaphoreType.DMA,
            pltpu.SemaphoreType.DMA,
            pltpu.SemaphoreType.DMA,
        ],
    )
    def pool_kernel(x_hbm, table_hbm, out_hbm, idx_v, rows_v, pooled_v,
                    sem0, sem1, sem2):
        wid = lax.axis_index("s") * NC + lax.axis_index("c")
        pltpu.sync_copy(x_hbm.at[wid], idx_v)

        sems = (sem0, sem1, sem2)

        def issue(r, b):
            for off, n in CHUNKS:
                pltpu.make_async_copy(
                    table_hbm.at[idx_v.at[r, pl.ds(off, n)]],
                    rows_v.at[b, pl.ds(off, n)],
                    sems[b],
                ).start()

        def wait(r, b):
            for off, n in CHUNKS:
                pltpu.make_async_copy(
                    table_hbm.at[idx_v.at[r, pl.ds(off, n)]],
                    rows_v.at[b, pl.ds(off, n)],
                    sems[b],
                ).wait()

        def reduce_row(r, b):
            rows = rows_v.at[b]

            def body(j, accs):
                return tuple(
                    accs[k] + rows[j, pl.ds(k * 16, 16)] for k in range(4)
                )

            init = tuple(jnp.zeros((16,), jnp.float32) for _ in range(4))
            accs = lax.fori_loop(0, L_SEQ, body, init, unroll=8)
            scale = jnp.float32(1.0 / L_SEQ)
            for k in range(4):
                pooled_v[r, pl.ds(k * 16, 16)] = accs[k] * scale

        issue(0, 0)
        issue(1, 1)
        n_main = (b_per_w - 2) // 3

        def outer(g, carry):
            for b in range(3):
                r = g * 3 + b
                issue(r + 2, (b + 2) % 3)
                wait(r, b)
                reduce_row(r, b)
            return carry

        lax.fori_loop(0, n_main, outer, 0)
        for r in range(3 * n_main, b_per_w):
            wait(r, r % 3)
            reduce_row(r, r % 3)
        pltpu.sync_copy(pooled_v, out_hbm.at[pl.ds(wid * b_per_w, b_per_w)])

    return pool_kernel


def _tail_kernel(p_ref, wt_ref, b_ref, g_ref, be_ref, o_ref):
    p = p_ref[...]
    h = jnp.dot(p, wt_ref[...], preferred_element_type=jnp.float32)
    h = h + b_ref[...]
    n = jnp.float32(1.0 / p.shape[0])
    mu = jnp.sum(h, axis=0, keepdims=True) * n
    d = h - mu
    var = jnp.sum(d * d, axis=0, keepdims=True) * n
    hn = d * lax.rsqrt(var + EPS) * g_ref[...] + be_ref[...]
    m = jnp.float32(1.0 / p.shape[1])
    mu2 = jnp.sum(hn, axis=1, keepdims=True) * m
    d2 = hn - mu2
    var2 = jnp.sum(d2 * d2, axis=1, keepdims=True) * m
    o_ref[...] = d2 * lax.rsqrt(var2 + EPS)


@jax.jit
def kernel(x, table, W, b, gamma, beta):
    B = x.shape[0]
    info = plsc.get_sparse_core_info()
    NW = info.num_cores * info.num_subcores
    x_pad = jnp.pad(x, ((0, 0), (0, L_PAD - L_SEQ)))
    x_blocks = x_pad.reshape(NW, B // NW, L_PAD)
    table_wide = _convert_table(table.T)
    pooled = _make_pooling(B)(x_blocks, table_wide)
    return pl.pallas_call(
        _tail_kernel,
        out_shape=jax.ShapeDtypeStruct((B, DIM), jnp.float32),
    )(pooled, W.T, b.reshape(1, DIM), gamma.reshape(1, DIM),
      beta.reshape(1, DIM))
